# baseline (device time: 30227 ns/iter reference)
import jax
import jax.numpy as jnp
from jax import lax
from jax.experimental import pallas as pl
from jax.experimental.pallas import tpu as pltpu

N_DEV = 8
BLK = 128

_NEAR_FIRST = (1, 3, 2, 4, 5, 6, 7)


def kernel(x, router_W, route_idx, expert_W, shared_W):
    n_tok, d = x.shape
    e_loc, _, h = expert_W.shape

    def body(x_ref, rw_ref, idx_ref, ew_ref, sw_ref, out_ref,
             comm_ref, stage_ref, gath_ref,
             rs_send, rs_recv, ag_send, ag_recv):
        my = lax.axis_index("i")

        barrier = pltpu.get_barrier_semaphore()
        for k in range(1, N_DEV):
            pl.semaphore_signal(
                barrier, inc=1, device_id=(jnp.bitwise_xor(my, k),),
                device_id_type=pl.DeviceIdType.MESH)
        pl.semaphore_wait(barrier, N_DEV - 1)

        xf = x_ref[:, :]
        xb = xf.astype(jnp.bfloat16)

        sc = jnp.dot(xb, rw_ref[:, :].astype(jnp.bfloat16),
                     preferred_element_type=jnp.float32)
        m = jnp.max(sc, axis=1, keepdims=True)
        p = jnp.exp(sc - m)
        p = p / jnp.sum(p, axis=1, keepdims=True)

        col = lax.broadcasted_iota(jnp.int32, p.shape, 1)
        idx = idx_ref[:, :]

        parts = []
        for j in range(e_loc):
            gid = my * e_loc + j
            pj = jnp.sum(jnp.where(col == gid, p, 0.0), axis=1,
                         keepdims=True)
            coef = jnp.where(idx == gid, pj, 0.0)
            parts.append((xf * coef).astype(jnp.bfloat16))
        xcat = jnp.concatenate(parts, axis=1)
        ewcat = jnp.concatenate(
            [ew_ref[j].astype(jnp.bfloat16) for j in range(e_loc)], axis=0)
        partial = jnp.dot(xcat, ewcat, preferred_element_type=jnp.float32)

        comm_ref[:, :] = partial.astype(jnp.bfloat16)

        rs = {}
        for k in range(N_DEV - 1, 0, -1):
            peer = jnp.bitwise_xor(my, k)
            rs[k] = pltpu.make_async_remote_copy(
                src_ref=comm_ref.at[pl.ds(peer * BLK, BLK)],
                dst_ref=stage_ref.at[k - 1],
                send_sem=rs_send.at[k - 1], recv_sem=rs_recv.at[k - 1],
                device_id=(peer,), device_id_type=pl.DeviceIdType.MESH)
            rs[k].start()

        shared = jnp.dot(xb, sw_ref[:, :].astype(jnp.bfloat16),
                         preferred_element_type=jnp.float32)
        out_ref[:, :] = shared

        red = comm_ref[pl.ds(my * BLK, BLK), :]
        for k in _NEAR_FIRST:
            rs[k].wait()
            red = red + stage_ref[k - 1, :, :]

        gath_ref[pl.ds(my * BLK, BLK), :] = red
        ag = {}
        for k in range(N_DEV - 1, 0, -1):
            ag[k] = pltpu.make_async_remote_copy(
                src_ref=gath_ref.at[pl.ds(my * BLK, BLK)],
                dst_ref=gath_ref.at[pl.ds(my * BLK, BLK)],
                send_sem=ag_send.at[k - 1], recv_sem=ag_recv.at[k - 1],
                device_id=(jnp.bitwise_xor(my, k),),
                device_id_type=pl.DeviceIdType.MESH)
            ag[k].start()

        out_ref[pl.ds(my * BLK, BLK), :] = (
            out_ref[pl.ds(my * BLK, BLK), :] + red.astype(jnp.float32))

        for k in _NEAR_FIRST:
            ag[k].wait()
            boff = jnp.bitwise_xor(my, k) * BLK
            out_ref[pl.ds(boff, BLK), :] = (
                out_ref[pl.ds(boff, BLK), :]
                + gath_ref[pl.ds(boff, BLK), :].astype(jnp.float32))

    return pl.pallas_call(
        body,
        out_shape=jax.ShapeDtypeStruct((n_tok, h), jnp.float32),
        in_specs=[pl.BlockSpec(memory_space=pltpu.VMEM)] * 5,
        out_specs=pl.BlockSpec(memory_space=pltpu.VMEM),
        scratch_shapes=[
            pltpu.VMEM((n_tok, h), jnp.bfloat16),
            pltpu.VMEM((N_DEV - 1, BLK, h), jnp.bfloat16),
            pltpu.VMEM((n_tok, h), jnp.bfloat16),
            pltpu.SemaphoreType.DMA((N_DEV - 1,)),
            pltpu.SemaphoreType.DMA((N_DEV - 1,)),
            pltpu.SemaphoreType.DMA((N_DEV - 1,)),
            pltpu.SemaphoreType.DMA((N_DEV - 1,)),
        ],
        compiler_params=pltpu.CompilerParams(collective_id=0),
    )(x, router_W, route_idx, expert_W, shared_W)


# device time: 24921 ns/iter; 1.2129x vs baseline; 1.2129x over previous
import jax
import jax.numpy as jnp
from jax import lax
from jax.experimental import pallas as pl
from jax.experimental.pallas import tpu as pltpu

N_DEV = 8
BLK = 128

CAP = 48

_MODE = 2

_NEAR_FIRST = (1, 3, 2, 4, 5, 6, 7)


def kernel(x, router_W, route_idx, expert_W, shared_W):
    n_tok, d = x.shape
    e_loc, _, h = expert_W.shape

    hh = h // 2

    def body(x_ref, rw_ref, idx_ref, ew_ref, sw_ref, out_ref,
             sbufa_ref, sbufb_ref, stga_ref, stgb_ref, gatha_ref, gathb_ref,
             rsa_send, rsa_recv, rsb_send, rsb_recv,
             aga_send, aga_recv, agb_send, agb_recv):
        my = lax.axis_index("i")

        if _MODE >= 1:
            barrier = pltpu.get_barrier_semaphore()
            for k in range(1, N_DEV):
                pl.semaphore_signal(
                    barrier, inc=1, device_id=(jnp.bitwise_xor(my, k),),
                    device_id_type=pl.DeviceIdType.MESH)

        xf = x_ref[:, :]
        xb = xf.astype(jnp.bfloat16)

        sc = jnp.dot(xb, rw_ref[:, :].astype(jnp.bfloat16),
                     preferred_element_type=jnp.float32)
        m = jnp.max(sc, axis=1, keepdims=True)
        p = jnp.exp(sc - m)
        p = p / jnp.sum(p, axis=1, keepdims=True)

        col = lax.broadcasted_iota(jnp.int32, p.shape, 1)
        idx = idx_ref[:, :]

        parts = []
        for j in range(e_loc):
            gid = my * e_loc + j
            pj = jnp.sum(jnp.where(col == gid, p, 0.0), axis=1,
                         keepdims=True)
            coef = jnp.where(idx == gid, pj, 0.0).astype(jnp.bfloat16)
            parts.append(xb * coef)
        xcat = jnp.concatenate(parts, axis=1)
        ewcat = jnp.concatenate(
            [ew_ref[j].astype(jnp.bfloat16) for j in range(e_loc)], axis=0)

        msk = ((idx >= my * e_loc) & (idx < (my + 1) * e_loc))
        tril = (lax.broadcasted_iota(jnp.int32, (BLK, BLK), 0)
                >= lax.broadcasted_iota(jnp.int32, (BLK, BLK), 1)
                ).astype(jnp.bfloat16)
        cid1 = (lax.broadcasted_iota(jnp.int32, (BLK, CAP), 1) + 1
                ).astype(jnp.float32)
        msk_bf = msk.astype(jnp.bfloat16)
        mcols = jnp.concatenate(
            [msk_bf[b * BLK:(b + 1) * BLK, :] for b in range(N_DEV)], axis=1)
        pos_all = jnp.dot(tril, mcols,
                          preferred_element_type=jnp.float32)
        qsbs = [
            jnp.where((pos_all[:, b:b + 1] == cid1)
                      & msk[b * BLK:(b + 1) * BLK, :], 1.0, 0.0
                      ).astype(jnp.bfloat16)
            for b in range(N_DEV)
        ]

        def rs_wave(partial_h, sbuf, stage, send_sems, recv_sems):
            for b in range(N_DEV):
                sbuf[pl.ds(b * CAP, CAP), :] = lax.dot_general(
                    qsbs[b], partial_h[b * BLK:(b + 1) * BLK, :],
                    (((0,), (0,)), ((), ())),
                    preferred_element_type=jnp.float32,
                ).astype(jnp.bfloat16)
            descs = {}
            if _MODE >= 1:
                for k in range(N_DEV - 1, 0, -1):
                    peer = jnp.bitwise_xor(my, k)
                    descs[k] = pltpu.make_async_remote_copy(
                        src_ref=sbuf.at[pl.ds(peer * CAP, CAP)],
                        dst_ref=stage.at[k],
                        send_sem=send_sems.at[k - 1],
                        recv_sem=recv_sems.at[k - 1],
                        device_id=(peer,),
                        device_id_type=pl.DeviceIdType.MESH)
                    descs[k].start()
            return descs

        def red_ag_wave(rsdescs, qr, sbuf, stage, gath, send_sems, recv_sems):
            stage[0, :, :] = sbuf[pl.ds(my * CAP, CAP), :]
            if _MODE >= 1:
                for k in _NEAR_FIRST:
                    rsdescs[k].wait()
            red = jnp.dot(qr, stage[:, :, :].reshape(N_DEV * CAP, hh),
                          preferred_element_type=jnp.float32
                          ).astype(jnp.bfloat16)
            gath[pl.ds(my * BLK, BLK), :] = red
            descs = {}
            if _MODE >= 2:
                for k in range(N_DEV - 1, 0, -1):
                    descs[k] = pltpu.make_async_remote_copy(
                        src_ref=gath.at[pl.ds(my * BLK, BLK)],
                        dst_ref=gath.at[pl.ds(my * BLK, BLK)],
                        send_sem=send_sems.at[k - 1],
                        recv_sem=recv_sems.at[k - 1],
                        device_id=(jnp.bitwise_xor(my, k),),
                        device_id_type=pl.DeviceIdType.MESH)
                    descs[k].start()
            return red, descs

        partial_a = jnp.dot(xcat, ewcat[:, :hh],
                            preferred_element_type=jnp.float32)
        if _MODE >= 1:
            pl.semaphore_wait(barrier, N_DEV - 1)
        rs_a = rs_wave(partial_a, sbufa_ref, stga_ref, rsa_send, rsa_recv)

        partial_b = jnp.dot(xcat, ewcat[:, hh:],
                            preferred_element_type=jnp.float32)
        rs_b = rs_wave(partial_b, sbufb_ref, stgb_ref, rsb_send, rsb_recv)

        idx_my = idx_ref[pl.ds(my * BLK, BLK), :]
        kk = jnp.bitwise_xor(idx_my // e_loc, my)
        onehot = (kk == lax.broadcasted_iota(jnp.int32, (BLK, N_DEV), 1)
                  ).astype(jnp.bfloat16)
        cnt = jnp.dot(tril, onehot, preferred_element_type=jnp.float32)
        rank = jnp.sum(cnt * onehot, axis=1, keepdims=True)
        jcol2 = lax.broadcasted_iota(jnp.int32, (BLK, N_DEV * CAP), 1)
        qr = jnp.where((kk == jcol2 // CAP)
                       & (rank == (jcol2 % CAP + 1).astype(jnp.float32)),
                       1.0, 0.0).astype(jnp.bfloat16)

        red_a, ag_a = red_ag_wave(rs_a, qr, sbufa_ref, stga_ref, gatha_ref,
                                  aga_send, aga_recv)

        shared = jnp.dot(xb, sw_ref[:, :].astype(jnp.bfloat16),
                         preferred_element_type=jnp.float32)
        out_ref[:, :] = shared

        red_b, ag_b = red_ag_wave(rs_b, qr, sbufb_ref, stgb_ref, gathb_ref,
                                  agb_send, agb_recv)

        out_ref[pl.ds(my * BLK, BLK), pl.ds(0, hh)] = (
            out_ref[pl.ds(my * BLK, BLK), pl.ds(0, hh)]
            + red_a.astype(jnp.float32))
        out_ref[pl.ds(my * BLK, BLK), pl.ds(hh, hh)] = (
            out_ref[pl.ds(my * BLK, BLK), pl.ds(hh, hh)]
            + red_b.astype(jnp.float32))

        if _MODE >= 2:
            for descs, gath, coff in ((ag_a, gatha_ref, 0),
                                      (ag_b, gathb_ref, hh)):
                for k in _NEAR_FIRST:
                    descs[k].wait()
                    boff = jnp.bitwise_xor(my, k) * BLK
                    out_ref[pl.ds(boff, BLK), pl.ds(coff, hh)] = (
                        out_ref[pl.ds(boff, BLK), pl.ds(coff, hh)]
                        + gath[pl.ds(boff, BLK), :].astype(jnp.float32))

    return pl.pallas_call(
        body,
        out_shape=jax.ShapeDtypeStruct((n_tok, h), jnp.float32),
        in_specs=[pl.BlockSpec(memory_space=pltpu.VMEM)] * 5,
        out_specs=pl.BlockSpec(memory_space=pltpu.VMEM),
        scratch_shapes=[
            pltpu.VMEM((N_DEV * CAP, h // 2), jnp.bfloat16),
            pltpu.VMEM((N_DEV * CAP, h // 2), jnp.bfloat16),
            pltpu.VMEM((N_DEV, CAP, h // 2), jnp.bfloat16),
            pltpu.VMEM((N_DEV, CAP, h // 2), jnp.bfloat16),
            pltpu.VMEM((n_tok, h // 2), jnp.bfloat16),
            pltpu.VMEM((n_tok, h // 2), jnp.bfloat16),
            pltpu.SemaphoreType.DMA((N_DEV - 1,)),
            pltpu.SemaphoreType.DMA((N_DEV - 1,)),
            pltpu.SemaphoreType.DMA((N_DEV - 1,)),
            pltpu.SemaphoreType.DMA((N_DEV - 1,)),
            pltpu.SemaphoreType.DMA((N_DEV - 1,)),
            pltpu.SemaphoreType.DMA((N_DEV - 1,)),
            pltpu.SemaphoreType.DMA((N_DEV - 1,)),
            pltpu.SemaphoreType.DMA((N_DEV - 1,)),
        ],
        compiler_params=(pltpu.CompilerParams(collective_id=0) if _MODE >= 1
                         else pltpu.CompilerParams()),
    )(x, router_W, route_idx, expert_W, shared_W)


# device time: 24553 ns/iter; 1.2311x vs baseline; 1.0150x over previous
import jax
import jax.numpy as jnp
from jax import lax
from jax.experimental import pallas as pl
from jax.experimental.pallas import tpu as pltpu

N_DEV = 8
BLK = 128

CAP = 48

_MODE = 2

_NEAR_FIRST = (1, 3, 2, 4, 5, 6, 7)


def kernel(x, router_W, route_idx, expert_W, shared_W):
    n_tok, d = x.shape
    e_loc, _, h = expert_W.shape

    hh = h // 2

    def body(x_ref, rw_ref, idx_ref, ew_ref, sw_ref, out_ref,
             sbufa_ref, sbufb_ref, stga_ref, stgb_ref, gatha_ref, gathb_ref,
             rsa_send, rsa_recv, rsb_send, rsb_recv,
             aga_send, aga_recv, agb_send, agb_recv):
        my = lax.axis_index("i")

        if _MODE >= 1:
            barrier = pltpu.get_barrier_semaphore()
            for k in range(1, N_DEV):
                pl.semaphore_signal(
                    barrier, inc=1, device_id=(jnp.bitwise_xor(my, k),),
                    device_id_type=pl.DeviceIdType.MESH)

        xf = x_ref[:, :]
        xb = xf.astype(jnp.bfloat16)

        sc = jnp.dot(xb, rw_ref[:, :].astype(jnp.bfloat16),
                     preferred_element_type=jnp.float32)
        m = jnp.max(sc, axis=1, keepdims=True)
        p = jnp.exp(sc - m)
        p = p / jnp.sum(p, axis=1, keepdims=True)

        col = lax.broadcasted_iota(jnp.int32, p.shape, 1)
        idx = idx_ref[:, :]

        coefsum = jnp.zeros((n_tok, 1), jnp.float32)
        for j in range(e_loc):
            gid = my * e_loc + j
            pj = jnp.sum(jnp.where(col == gid, p, 0.0), axis=1,
                         keepdims=True)
            coefsum = coefsum + jnp.where(idx == gid, pj, 0.0)
        ewb = [ew_ref[j].astype(jnp.bfloat16) for j in range(e_loc)]

        msk = ((idx >= my * e_loc) & (idx < (my + 1) * e_loc))
        tril = (lax.broadcasted_iota(jnp.int32, (BLK, BLK), 0)
                >= lax.broadcasted_iota(jnp.int32, (BLK, BLK), 1)
                ).astype(jnp.bfloat16)
        cid1 = (lax.broadcasted_iota(jnp.int32, (BLK, CAP), 1) + 1
                ).astype(jnp.float32)
        msk_bf = msk.astype(jnp.bfloat16)
        mcols = jnp.concatenate(
            [msk_bf[b * BLK:(b + 1) * BLK, :] for b in range(N_DEV)], axis=1)
        pos_all = jnp.dot(tril, mcols,
                          preferred_element_type=jnp.float32)
        qsbs = [
            jnp.where((pos_all[:, b:b + 1] == cid1)
                      & msk[b * BLK:(b + 1) * BLK, :], 1.0, 0.0
                      ).astype(jnp.bfloat16)
            for b in range(N_DEV)
        ]

        coef_bf = coefsum.astype(jnp.bfloat16)
        sel_parts = []
        idx_parts = []
        idx_f = idx.astype(jnp.float32)
        for b in range(N_DEV):
            sb = qsbs[b] * coef_bf[b * BLK:(b + 1) * BLK, :]
            sel_parts.append(lax.dot_general(
                sb, xb[b * BLK:(b + 1) * BLK, :],
                (((0,), (0,)), ((), ())),
                preferred_element_type=jnp.float32).astype(jnp.bfloat16))
            idx_parts.append(lax.dot_general(
                qsbs[b], idx_f[b * BLK:(b + 1) * BLK, :],
                (((0,), (0,)), ((), ())),
                preferred_element_type=jnp.float32))
        selxc = jnp.concatenate(sel_parts, axis=0)
        selidx = jnp.concatenate(idx_parts, axis=0)
        mparts = [
            selxc * (selidx == (my * e_loc + j).astype(jnp.float32)
                     ).astype(jnp.bfloat16)
            for j in range(e_loc)
        ]

        def expert_half(c0):
            acc = jnp.dot(mparts[0], ewb[0][:, c0:c0 + hh],
                          preferred_element_type=jnp.float32)
            for j in range(1, e_loc):
                acc = acc + jnp.dot(mparts[j], ewb[j][:, c0:c0 + hh],
                                    preferred_element_type=jnp.float32)
            return acc

        def rs_wave(compact_h, sbuf, stage, send_sems, recv_sems):
            sbuf[:, :] = compact_h.astype(jnp.bfloat16)
            descs = {}
            if _MODE >= 1:
                for k in range(N_DEV - 1, 0, -1):
                    peer = jnp.bitwise_xor(my, k)
                    descs[k] = pltpu.make_async_remote_copy(
                        src_ref=sbuf.at[pl.ds(peer * CAP, CAP)],
                        dst_ref=stage.at[k],
                        send_sem=send_sems.at[k - 1],
                        recv_sem=recv_sems.at[k - 1],
                        device_id=(peer,),
                        device_id_type=pl.DeviceIdType.MESH)
                    descs[k].start()
            return descs

        def red_ag_wave(rsdescs, qr, sbuf, stage, gath, send_sems, recv_sems):
            stage[0, :, :] = sbuf[pl.ds(my * CAP, CAP), :]
            if _MODE >= 1:
                for k in _NEAR_FIRST:
                    rsdescs[k].wait_recv()
            red = jnp.dot(qr, stage[:, :, :].reshape(N_DEV * CAP, hh),
                          preferred_element_type=jnp.float32
                          ).astype(jnp.bfloat16)
            gath[pl.ds(my * BLK, BLK), :] = red
            descs = {}
            if _MODE >= 2:
                for k in range(N_DEV - 1, 0, -1):
                    descs[k] = pltpu.make_async_remote_copy(
                        src_ref=gath.at[pl.ds(my * BLK, BLK)],
                        dst_ref=gath.at[pl.ds(my * BLK, BLK)],
                        send_sem=send_sems.at[k - 1],
                        recv_sem=recv_sems.at[k - 1],
                        device_id=(jnp.bitwise_xor(my, k),),
                        device_id_type=pl.DeviceIdType.MESH)
                    descs[k].start()
            return red, descs

        compact_a = expert_half(0)
        if _MODE >= 1:
            pl.semaphore_wait(barrier, N_DEV - 1)
        rs_a = rs_wave(compact_a, sbufa_ref, stga_ref, rsa_send, rsa_recv)

        compact_b = expert_half(hh)
        rs_b = rs_wave(compact_b, sbufb_ref, stgb_ref, rsb_send, rsb_recv)

        idx_my = idx_ref[pl.ds(my * BLK, BLK), :]
        kk = jnp.bitwise_xor(idx_my // e_loc, my)
        onehot = (kk == lax.broadcasted_iota(jnp.int32, (BLK, N_DEV), 1)
                  ).astype(jnp.bfloat16)
        cnt = jnp.dot(tril, onehot, preferred_element_type=jnp.float32)
        rank = jnp.sum(cnt * onehot, axis=1, keepdims=True)
        jcol2 = lax.broadcasted_iota(jnp.int32, (BLK, N_DEV * CAP), 1)
        qr = jnp.where((kk == jcol2 // CAP)
                       & (rank == (jcol2 % CAP + 1).astype(jnp.float32)),
                       1.0, 0.0).astype(jnp.bfloat16)

        red_a, ag_a = red_ag_wave(rs_a, qr, sbufa_ref, stga_ref, gatha_ref,
                                  aga_send, aga_recv)

        shared = jnp.dot(xb, sw_ref[:, :].astype(jnp.bfloat16),
                         preferred_element_type=jnp.float32)
        out_ref[:, :] = shared

        red_b, ag_b = red_ag_wave(rs_b, qr, sbufb_ref, stgb_ref, gathb_ref,
                                  agb_send, agb_recv)

        out_ref[pl.ds(my * BLK, BLK), pl.ds(0, hh)] = (
            out_ref[pl.ds(my * BLK, BLK), pl.ds(0, hh)]
            + red_a.astype(jnp.float32))
        out_ref[pl.ds(my * BLK, BLK), pl.ds(hh, hh)] = (
            out_ref[pl.ds(my * BLK, BLK), pl.ds(hh, hh)]
            + red_b.astype(jnp.float32))

        if _MODE >= 2:
            for descs, gath, coff in ((ag_a, gatha_ref, 0),
                                      (ag_b, gathb_ref, hh)):
                for k in _NEAR_FIRST:
                    descs[k].wait_recv()
                    boff = jnp.bitwise_xor(my, k) * BLK
                    out_ref[pl.ds(boff, BLK), pl.ds(coff, hh)] = (
                        out_ref[pl.ds(boff, BLK), pl.ds(coff, hh)]
                        + gath[pl.ds(boff, BLK), :].astype(jnp.float32))

        if _MODE >= 1:
            for k in range(1, N_DEV):
                rs_a[k].wait_send()
                rs_b[k].wait_send()
        if _MODE >= 2:
            for k in range(1, N_DEV):
                ag_a[k].wait_send()
                ag_b[k].wait_send()

    return pl.pallas_call(
        body,
        out_shape=jax.ShapeDtypeStruct((n_tok, h), jnp.float32),
        in_specs=[pl.BlockSpec(memory_space=pltpu.VMEM)] * 5,
        out_specs=pl.BlockSpec(memory_space=pltpu.VMEM),
        scratch_shapes=[
            pltpu.VMEM((N_DEV * CAP, h // 2), jnp.bfloat16),
            pltpu.VMEM((N_DEV * CAP, h // 2), jnp.bfloat16),
            pltpu.VMEM((N_DEV, CAP, h // 2), jnp.bfloat16),
            pltpu.VMEM((N_DEV, CAP, h // 2), jnp.bfloat16),
            pltpu.VMEM((n_tok, h // 2), jnp.bfloat16),
            pltpu.VMEM((n_tok, h // 2), jnp.bfloat16),
            pltpu.SemaphoreType.DMA((N_DEV - 1,)),
            pltpu.SemaphoreType.DMA((N_DEV - 1,)),
            pltpu.SemaphoreType.DMA((N_DEV - 1,)),
            pltpu.SemaphoreType.DMA((N_DEV - 1,)),
            pltpu.SemaphoreType.DMA((N_DEV - 1,)),
            pltpu.SemaphoreType.DMA((N_DEV - 1,)),
            pltpu.SemaphoreType.DMA((N_DEV - 1,)),
            pltpu.SemaphoreType.DMA((N_DEV - 1,)),
        ],
        compiler_params=(pltpu.CompilerParams(collective_id=0) if _MODE >= 1
                         else pltpu.CompilerParams()),
    )(x, router_W, route_idx, expert_W, shared_W)
